# baseline (speedup 1.0000x reference)
import functools

import jax
import jax.numpy as jnp
from jax import lax
from jax.experimental import pallas as pl
from jax.experimental.pallas import tpu as pltpu
from jax.experimental.pallas import tpu_sc as plsc

N = 10000
NP = 10240
E = 160000
EP = 163840
NC = 2
NS = 16
CHUNK = 128
ECHUNKS = EP // CHUNK
D_IN = 256
D_HID = 512
D_OUT = 256
HALF = 128
WIN = 2560
NPASS = NP // WIN
WPT = WIN // NS

_sc_mesh = plsc.VectorSubcoreMesh(core_axis_name="c", subcore_axis_name="s")


def _build_deg_kernel():
    nchunks = ECHUNKS // (NC * NS)

    @functools.partial(
        pl.kernel,
        out_type=jax.ShapeDtypeStruct((NC * NS, NP), jnp.float32),
        mesh=_sc_mesh,
        scratch_types=[
            pltpu.VMEM((nchunks * CHUNK,), jnp.int32),
            pltpu.VMEM((NP,), jnp.float32),
        ],
        compiler_params=pltpu.CompilerParams(needs_layout_passes=False),
    )
    def deg_kernel(dst1_hbm, zeros_hbm, degp_hbm, dstv, hist):
        c = lax.axis_index("c")
        s = lax.axis_index("s")
        wid = c * NS + s
        pltpu.sync_copy(dst1_hbm.at[pl.ds(wid * nchunks * CHUNK, nchunks * CHUNK)], dstv)
        pltpu.sync_copy(zeros_hbm, hist)
        ones16 = jnp.ones((16,), jnp.float32)

        def body(j, _):
            for k in range(8):
                plsc.addupdate_scatter(
                    hist, [dstv[pl.ds(j * CHUNK + k * 16, 16)]], ones16)
            return ()

        lax.fori_loop(0, nchunks, body, ())
        pltpu.sync_copy(hist, degp_hbm.at[wid])

    return deg_kernel


def _build_agg_kernel():
    nchunks = ECHUNKS // NS
    rpt = NP // NS

    @functools.partial(
        pl.kernel,
        out_type=jax.ShapeDtypeStruct((NC * NP, HALF), jnp.float32),
        mesh=_sc_mesh,
        scratch_types=[
            pltpu.VMEM((nchunks * CHUNK,), jnp.int32),
            pltpu.VMEM((CHUNK, HALF), jnp.float32),
            pltpu.VMEM((CHUNK, HALF), jnp.float32),
            pltpu.VMEM((CHUNK,), jnp.int32),
            pltpu.VMEM((CHUNK,), jnp.int32),
            pltpu.VMEM((CHUNK,), jnp.int32),
            pltpu.VMEM((CHUNK,), jnp.int32),
            pltpu.VMEM_SHARED((NP, HALF), jnp.float32),
            pltpu.SemaphoreType.DMA,
            pltpu.SemaphoreType.DMA,
            pltpu.SemaphoreType.DMA,
            pltpu.SemaphoreType.DMA,
        ],
    )
    def agg_kernel(vh_hbm, src1_hbm, dst2_hbm, out_hbm,
                   srcv, rowsa, rowsb, gidxa, gidxb, sidxa, sidxb,
                   acc, sema, semb, semsa, semsb):
        c = lax.axis_index("c")
        s = lax.axis_index("s")
        goff = c * NP
        ne = nchunks * CHUNK
        pltpu.sync_copy(src1_hbm.at[pl.ds(s * ne, ne)], srcv)
        pltpu.sync_copy(
            vh_hbm.at[pl.ds(goff + s * rpt, rpt)],
            acc.at[pl.ds(s * rpt, rpt)],
        )
        plsc.subcore_barrier()

        def fire_g(gidx, sidx, rows, sem, j):
            for k in range(8):
                gidx[pl.ds(k * 16, 16)] = srcv[pl.ds(j * CHUNK + k * 16, 16)] + goff
            pltpu.async_copy(vh_hbm.at[gidx], rows, sem)
            pltpu.async_copy(dst2_hbm.at[s * nchunks + j], sidx, sem)

        def scat(gidx, sidx, rows, sem, sems):
            pltpu.make_async_copy(vh_hbm.at[gidx], rows, sem).wait()
            pltpu.make_async_copy(dst2_hbm.at[0], sidx, sem).wait()
            pltpu.async_copy(rows, acc.at[sidx], sems, add=True)

        def wait_s(sidx, rows, sems):
            pltpu.make_async_copy(rows, acc.at[sidx], sems).wait()

        fire_g(gidxa, sidxa, rowsa, sema, 0)
        fire_g(gidxb, sidxb, rowsb, semb, 1)

        def body(i, _):
            a = 2 * i
            scat(gidxa, sidxa, rowsa, sema, semsa)
            scat(gidxb, sidxb, rowsb, semb, semsb)
            wait_s(sidxa, rowsa, semsa)
            fire_g(gidxa, sidxa, rowsa, sema, a + 2)
            wait_s(sidxb, rowsb, semsb)
            fire_g(gidxb, sidxb, rowsb, semb, a + 3)
            return ()

        lax.fori_loop(0, nchunks // 2 - 1, body, ())
        scat(gidxa, sidxa, rowsa, sema, semsa)
        scat(gidxb, sidxb, rowsb, semb, semsb)
        wait_s(sidxa, rowsa, semsa)
        wait_s(sidxb, rowsb, semsb)
        plsc.subcore_barrier()
        pltpu.sync_copy(
            acc.at[pl.ds(s * rpt, rpt)],
            out_hbm.at[pl.ds(goff + s * rpt, rpt)],
        )

    return agg_kernel


_deg = _build_deg_kernel()
_agg = _build_agg_kernel()


_RT = 512
_GRID = NP // _RT


def _prescale_body(degp_ref, x_ref, vh_ref, dinv_ref):
    deg = jnp.sum(degp_ref[...], axis=0) + 1.0
    dinv = lax.rsqrt(deg)
    dinv_ref[...] = dinv.reshape(_RT, 1)
    vh = x_ref[...] * dinv[:, None]
    vh_ref[0] = vh[:, :HALF]
    vh_ref[1] = vh[:, HALF:]


def _prescale(degp, x_p):
    return pl.pallas_call(
        _prescale_body,
        grid=(_GRID,),
        in_specs=[
            pl.BlockSpec((NC * NS, _RT), lambda i: (0, i)),
            pl.BlockSpec((_RT, D_IN), lambda i: (i, 0)),
        ],
        out_specs=[
            pl.BlockSpec((NC, _RT, HALF), lambda i: (0, i, 0)),
            pl.BlockSpec((_RT, 1), lambda i: (i, 0)),
        ],
        out_shape=[
            jax.ShapeDtypeStruct((NC, NP, HALF), jnp.float32),
            jax.ShapeDtypeStruct((NP, 1), jnp.float32),
        ],
    )(degp, x_p)


def _mlp_body(s1_ref, dinv_ref, w1_ref, b1_ref, w2_ref, vh2_ref):
    dv = dinv_ref[...]
    w1 = w1_ref[...]
    z0 = s1_ref[0] * dv
    z1 = s1_ref[1] * dv
    h = jnp.dot(z0, w1[:HALF], preferred_element_type=jnp.float32)
    h = h + jnp.dot(z1, w1[HALF:], preferred_element_type=jnp.float32)
    h = jnp.maximum(h + b1_ref[...], 0.0)
    p = jnp.dot(h, w2_ref[...], preferred_element_type=jnp.float32) * dv
    vh2_ref[0] = p[:, :HALF]
    vh2_ref[1] = p[:, HALF:]


def _mlp(s1, dinv2d, W1, b1, W2):
    return pl.pallas_call(
        _mlp_body,
        grid=(_GRID,),
        in_specs=[
            pl.BlockSpec((NC, _RT, HALF), lambda i: (0, i, 0)),
            pl.BlockSpec((_RT, 1), lambda i: (i, 0)),
            pl.BlockSpec((D_IN, D_HID), lambda i: (0, 0)),
            pl.BlockSpec((1, D_HID), lambda i: (0, 0)),
            pl.BlockSpec((D_HID, D_OUT), lambda i: (0, 0)),
        ],
        out_specs=pl.BlockSpec((NC, _RT, HALF), lambda i: (0, i, 0)),
        out_shape=jax.ShapeDtypeStruct((NC, NP, HALF), jnp.float32),
    )(s1, dinv2d, W1, b1, W2)


def _final_body(s2_ref, dinv_ref, b2_ref, out_ref):
    dv = dinv_ref[...]
    s2 = jnp.concatenate([s2_ref[0], s2_ref[1]], axis=1)
    out_ref[...] = s2 * dv + b2_ref[...]


def _final(s2, dinv2d, b2r):
    return pl.pallas_call(
        _final_body,
        grid=(_GRID,),
        in_specs=[
            pl.BlockSpec((NC, _RT, HALF), lambda i: (0, i, 0)),
            pl.BlockSpec((_RT, 1), lambda i: (i, 0)),
            pl.BlockSpec((1, D_OUT), lambda i: (0, 0)),
        ],
        out_specs=pl.BlockSpec((_RT, D_OUT), lambda i: (i, 0)),
        out_shape=jax.ShapeDtypeStruct((N, D_OUT), jnp.float32),
    )(s2, dinv2d, b2r)


def kernel(x, edge_index, W1, b1, W2, b2):
    ei = edge_index.astype(jnp.int32)
    src = jnp.concatenate([ei[0], jnp.zeros((EP - E,), jnp.int32)])
    dst = jnp.concatenate([ei[1], jnp.full((EP - E,), N, jnp.int32)])

    x_p = jnp.pad(x, ((0, NP - N), (0, 0)))
    zeros_np = jnp.zeros((NP,), jnp.float32)

    dst2 = dst.reshape(ECHUNKS, CHUNK)
    degp = _deg(dst, zeros_np)
    vh1, dinv2d = _prescale(degp, x_p)
    s1 = _agg(vh1.reshape(NC * NP, HALF), src, dst2).reshape(NC, NP, HALF)
    vh2 = _mlp(s1, dinv2d, W1, b1.reshape(1, D_HID), W2)
    s2 = _agg(vh2.reshape(NC * NP, HALF), src, dst2).reshape(NC, NP, HALF)
    out = _final(s2, dinv2d, b2.reshape(1, D_OUT))
    return out


# speedup vs baseline: 1.0951x; 1.0951x over previous
import functools

import jax
import jax.numpy as jnp
from jax import lax
from jax.experimental import pallas as pl
from jax.experimental.pallas import tpu as pltpu
from jax.experimental.pallas import tpu_sc as plsc

N = 10000
NP = 10240
E = 160000
EP = 163840
NC = 2
NS = 16
CHUNK = 128
ECHUNKS = EP // CHUNK
D_IN = 256
D_HID = 512
D_OUT = 256
HALF = 128
WIN = 2560
NPASS = NP // WIN
WPT = WIN // NS

_sc_mesh = plsc.VectorSubcoreMesh(core_axis_name="c", subcore_axis_name="s")


def _build_deg_kernel():
    nchunks = ECHUNKS // (NC * NS)

    @functools.partial(
        pl.kernel,
        out_type=jax.ShapeDtypeStruct((NC * NS, NP), jnp.float32),
        mesh=_sc_mesh,
        scratch_types=[
            pltpu.VMEM((nchunks * CHUNK,), jnp.int32),
            pltpu.VMEM((NP,), jnp.float32),
        ],
        compiler_params=pltpu.CompilerParams(needs_layout_passes=False),
    )
    def deg_kernel(dst1_hbm, zeros_hbm, degp_hbm, dstv, hist):
        c = lax.axis_index("c")
        s = lax.axis_index("s")
        wid = c * NS + s
        pltpu.sync_copy(dst1_hbm.at[pl.ds(wid * nchunks * CHUNK, nchunks * CHUNK)], dstv)
        pltpu.sync_copy(zeros_hbm, hist)
        ones16 = jnp.ones((16,), jnp.float32)

        def body(j, _):
            for k in range(8):
                plsc.addupdate_scatter(
                    hist, [dstv[pl.ds(j * CHUNK + k * 16, 16)]], ones16)
            return ()

        lax.fori_loop(0, nchunks, body, ())
        pltpu.sync_copy(hist, degp_hbm.at[wid])

    return deg_kernel


def _build_agg_kernel():
    nchunks = ECHUNKS // NS
    rpt = NP // NS

    @functools.partial(
        pl.kernel,
        out_type=jax.ShapeDtypeStruct((NC * NP, HALF), jnp.float32),
        mesh=_sc_mesh,
        scratch_types=[
            pltpu.VMEM((nchunks * CHUNK,), jnp.int32),
            pltpu.VMEM((CHUNK, HALF), jnp.float32),
            pltpu.VMEM((CHUNK, HALF), jnp.float32),
            pltpu.VMEM((CHUNK,), jnp.int32),
            pltpu.VMEM((CHUNK,), jnp.int32),
            pltpu.VMEM((CHUNK,), jnp.int32),
            pltpu.VMEM((CHUNK,), jnp.int32),
            pltpu.VMEM_SHARED((NP, HALF), jnp.float32),
            pltpu.SemaphoreType.DMA,
            pltpu.SemaphoreType.DMA,
            pltpu.SemaphoreType.DMA,
            pltpu.SemaphoreType.DMA,
        ],
    )
    def agg_kernel(vh_hbm, src1_hbm, dst2_hbm, out_hbm,
                   srcv, rowsa, rowsb, gidxa, gidxb, sidxa, sidxb,
                   acc, sema, semb, semsa, semsb):
        c = lax.axis_index("c")
        s = lax.axis_index("s")
        goff = c * NP
        ne = nchunks * CHUNK
        pltpu.sync_copy(src1_hbm.at[pl.ds(s * ne, ne)], srcv)
        pltpu.sync_copy(
            vh_hbm.at[pl.ds(goff + s * rpt, rpt)],
            acc.at[pl.ds(s * rpt, rpt)],
        )
        plsc.subcore_barrier()

        def fire_g(gidx, sidx, rows, sem, j):
            for k in range(8):
                gidx[pl.ds(k * 16, 16)] = srcv[pl.ds(j * CHUNK + k * 16, 16)] + goff
            pltpu.async_copy(vh_hbm.at[gidx], rows, sem)
            pltpu.async_copy(dst2_hbm.at[s * nchunks + j], sidx, sem)

        def drain(gidx, sidx, rows, sem):
            pltpu.make_async_copy(vh_hbm.at[gidx], rows, sem).wait()
            pltpu.make_async_copy(dst2_hbm.at[0], sidx, sem).wait()
            pltpu.sync_copy(rows, acc.at[sidx], add=True)

        fire_g(gidxa, sidxa, rowsa, sema, 0)

        def body(i, _):
            a = 2 * i
            fire_g(gidxb, sidxb, rowsb, semb, a + 1)
            drain(gidxa, sidxa, rowsa, sema)
            fire_g(gidxa, sidxa, rowsa, sema, a + 2)
            drain(gidxb, sidxb, rowsb, semb)
            return ()

        lax.fori_loop(0, nchunks // 2 - 1, body, ())
        fire_g(gidxb, sidxb, rowsb, semb, nchunks - 1)
        drain(gidxa, sidxa, rowsa, sema)
        drain(gidxb, sidxb, rowsb, semb)
        plsc.subcore_barrier()
        pltpu.sync_copy(
            acc.at[pl.ds(s * rpt, rpt)],
            out_hbm.at[pl.ds(goff + s * rpt, rpt)],
        )

    return agg_kernel


_deg = _build_deg_kernel()
_agg = _build_agg_kernel()


_RT = 512
_GRID = NP // _RT


def _prescale_body(degp_ref, x_ref, vh_ref, dinv_ref):
    deg = jnp.sum(degp_ref[...], axis=0) + 1.0
    dinv = lax.rsqrt(deg)
    dinv_ref[...] = dinv.reshape(_RT, 1)
    vh = x_ref[...] * dinv[:, None]
    vh_ref[0] = vh[:, :HALF]
    vh_ref[1] = vh[:, HALF:]


def _prescale(degp, x_p):
    return pl.pallas_call(
        _prescale_body,
        grid=(_GRID,),
        in_specs=[
            pl.BlockSpec((NC * NS, _RT), lambda i: (0, i)),
            pl.BlockSpec((_RT, D_IN), lambda i: (i, 0)),
        ],
        out_specs=[
            pl.BlockSpec((NC, _RT, HALF), lambda i: (0, i, 0)),
            pl.BlockSpec((_RT, 1), lambda i: (i, 0)),
        ],
        out_shape=[
            jax.ShapeDtypeStruct((NC, NP, HALF), jnp.float32),
            jax.ShapeDtypeStruct((NP, 1), jnp.float32),
        ],
    )(degp, x_p)


def _mlp_body(s1_ref, dinv_ref, w1_ref, b1_ref, w2_ref, vh2_ref):
    dv = dinv_ref[...]
    w1 = w1_ref[...]
    z0 = s1_ref[0] * dv
    z1 = s1_ref[1] * dv
    h = jnp.dot(z0, w1[:HALF], preferred_element_type=jnp.float32)
    h = h + jnp.dot(z1, w1[HALF:], preferred_element_type=jnp.float32)
    h = jnp.maximum(h + b1_ref[...], 0.0)
    p = jnp.dot(h, w2_ref[...], preferred_element_type=jnp.float32) * dv
    vh2_ref[0] = p[:, :HALF]
    vh2_ref[1] = p[:, HALF:]


def _mlp(s1, dinv2d, W1, b1, W2):
    return pl.pallas_call(
        _mlp_body,
        grid=(_GRID,),
        in_specs=[
            pl.BlockSpec((NC, _RT, HALF), lambda i: (0, i, 0)),
            pl.BlockSpec((_RT, 1), lambda i: (i, 0)),
            pl.BlockSpec((D_IN, D_HID), lambda i: (0, 0)),
            pl.BlockSpec((1, D_HID), lambda i: (0, 0)),
            pl.BlockSpec((D_HID, D_OUT), lambda i: (0, 0)),
        ],
        out_specs=pl.BlockSpec((NC, _RT, HALF), lambda i: (0, i, 0)),
        out_shape=jax.ShapeDtypeStruct((NC, NP, HALF), jnp.float32),
    )(s1, dinv2d, W1, b1, W2)


def _final_body(s2_ref, dinv_ref, b2_ref, out_ref):
    dv = dinv_ref[...]
    s2 = jnp.concatenate([s2_ref[0], s2_ref[1]], axis=1)
    out_ref[...] = s2 * dv + b2_ref[...]


def _final(s2, dinv2d, b2r):
    return pl.pallas_call(
        _final_body,
        grid=(_GRID,),
        in_specs=[
            pl.BlockSpec((NC, _RT, HALF), lambda i: (0, i, 0)),
            pl.BlockSpec((_RT, 1), lambda i: (i, 0)),
            pl.BlockSpec((1, D_OUT), lambda i: (0, 0)),
        ],
        out_specs=pl.BlockSpec((_RT, D_OUT), lambda i: (i, 0)),
        out_shape=jax.ShapeDtypeStruct((N, D_OUT), jnp.float32),
    )(s2, dinv2d, b2r)


def kernel(x, edge_index, W1, b1, W2, b2):
    ei = edge_index.astype(jnp.int32)
    src = jnp.concatenate([ei[0], jnp.zeros((EP - E,), jnp.int32)])
    dst = jnp.concatenate([ei[1], jnp.full((EP - E,), N, jnp.int32)])

    x_p = jnp.pad(x, ((0, NP - N), (0, 0)))
    zeros_np = jnp.zeros((NP,), jnp.float32)

    dst2 = dst.reshape(ECHUNKS, CHUNK)
    degp = _deg(dst, zeros_np)
    vh1, dinv2d = _prescale(degp, x_p)
    s1 = _agg(vh1.reshape(NC * NP, HALF), src, dst2).reshape(NC, NP, HALF)
    vh2 = _mlp(s1, dinv2d, W1, b1.reshape(1, D_HID), W2)
    s2 = _agg(vh2.reshape(NC * NP, HALF), src, dst2).reshape(NC, NP, HALF)
    out = _final(s2, dinv2d, b2.reshape(1, D_OUT))
    return out
